# baseline (speedup 1.0000x reference)
import jax
import jax.numpy as jnp
from jax.experimental import pallas as pl

C = 256
H = 100
W = 100
Z = 8
H_BLK = 8
W_BLK = 56


def _body(col_ref, row_ref, z_ref, out_ref):
    col_b = col_ref[...]
    row_b = row_ref[...]
    z_b = z_ref[...]
    out_ref[...] = (
        col_b[:, None, None, :] + row_b[None, :, None, :] + z_b[None, None, :, :]
    )


def kernel(row_weight, col_weight, z_weight, bs, h, w, z):
    pos4 = pl.pallas_call(
        _body,
        grid=(pl.cdiv(H, H_BLK), pl.cdiv(W, W_BLK)),
        in_specs=[
            pl.BlockSpec((H_BLK, C), lambda i, j: (i, 0)),
            pl.BlockSpec((W_BLK, C), lambda i, j: (j, 0)),
            pl.BlockSpec((Z, C), lambda i, j: (0, 0)),
        ],
        out_specs=pl.BlockSpec((H_BLK, W_BLK, Z, C), lambda i, j: (i, j, 0, 0)),
        out_shape=jax.ShapeDtypeStruct((H, W, Z, C), jnp.float32),
    )(col_weight, row_weight, z_weight)
    return jnp.transpose(pos4, (3, 0, 1, 2))[None]


# speedup vs baseline: 1.1952x; 1.1952x over previous
import jax
import jax.numpy as jnp
from jax.experimental import pallas as pl

C = 256
H = 100
W = 100
Z = 8
H_BLK = 8


def _body(col_ref, row_ref, z_ref, out_ref):
    col_b = col_ref[...]
    row_b = row_ref[...]
    z_b = z_ref[...]
    out_ref[...] = (
        col_b[:, None, None, :] + row_b[None, :, None, :] + z_b[None, None, :, :]
    )


def kernel(row_weight, col_weight, z_weight, bs, h, w, z):
    pos4 = pl.pallas_call(
        _body,
        grid=(pl.cdiv(H, H_BLK),),
        in_specs=[
            pl.BlockSpec((H_BLK, C), lambda i: (i, 0)),
            pl.BlockSpec((W, C), lambda i: (0, 0)),
            pl.BlockSpec((Z, C), lambda i: (0, 0)),
        ],
        out_specs=pl.BlockSpec((H_BLK, W, Z, C), lambda i: (i, 0, 0, 0)),
        out_shape=jax.ShapeDtypeStruct((H, W, Z, C), jnp.float32),
    )(col_weight, row_weight, z_weight)
    return jnp.transpose(pos4, (3, 0, 1, 2))[None]
